# baseline (speedup 1.0000x reference)
import functools

import jax
import jax.numpy as jnp
from jax import lax
from jax.experimental import pallas as pl
from jax.experimental.pallas import tpu as pltpu
from jax.experimental.pallas import tpu_sc as plsc

B, N, DIM = 16, 1024, 256
BN = B * N
K = 1024
M = 512
NB = BN // M

NC, NS = 2, 16
NW = NC * NS
BPW = BN // NW
CH = 128
NCH = BPW // CH


def _tc_body(x_ref, e_ref, dist_ref, idx_ref):
    x = x_ref[...]
    e = e_ref[...]
    cross = jax.lax.dot_general(
        x, e, (((1,), (1,)), ((), ())), preferred_element_type=jnp.float32
    )
    x_sq = jnp.sum(x * x, axis=1, keepdims=True)
    e_sq = jnp.sum(e * e, axis=1)[None, :]
    dist = x_sq + e_sq - 2.0 * cross
    dist_ref[...] = dist
    idx = jnp.argmin(dist, axis=1).astype(jnp.int32)
    idx_ref[...] = idx.reshape(1, 1, M)


@functools.partial(
    pl.kernel,
    mesh=plsc.VectorSubcoreMesh(core_axis_name="c", subcore_axis_name="s"),
    out_type=jax.ShapeDtypeStruct((BN, DIM), jnp.float32),
    scratch_types=[
        pltpu.VMEM((CH,), jnp.int32),
        pltpu.VMEM((CH, DIM), jnp.float32),
        pltpu.SemaphoreType.DMA,
    ],
)
def _sc_gather(idx_hbm, e_hbm, out_hbm, idx_v, rows_v, sem):
    wid = lax.axis_index("s") * NC + lax.axis_index("c")
    base = wid * BPW
    for c in range(NCH):
        off = base + c * CH
        pltpu.sync_copy(idx_hbm.at[pl.ds(off, CH)], idx_v)
        pltpu.async_copy(e_hbm.at[idx_v], rows_v, sem).wait()
        pltpu.sync_copy(rows_v, out_hbm.at[pl.ds(off, CH)])


def kernel(x, embed):
    xf = x.reshape(BN, DIM)
    e = embed.reshape(K, DIM)
    dist, idx3 = pl.pallas_call(
        _tc_body,
        grid=(NB,),
        in_specs=[
            pl.BlockSpec((M, DIM), lambda i: (i, 0)),
            pl.BlockSpec((K, DIM), lambda i: (0, 0)),
        ],
        out_specs=[
            pl.BlockSpec((M, K), lambda i: (i, 0)),
            pl.BlockSpec((1, 1, M), lambda i: (i, 0, 0)),
        ],
        out_shape=[
            jax.ShapeDtypeStruct((BN, K), jnp.float32),
            jax.ShapeDtypeStruct((NB, 1, M), jnp.int32),
        ],
    )(xf, e)
    idx = idx3.reshape(BN)
    q = _sc_gather(idx, e)
    return q.reshape(BN, 1, DIM), idx, dist


# speedup vs baseline: 1.0096x; 1.0096x over previous
import functools

import jax
import jax.numpy as jnp
from jax import lax
from jax.experimental import pallas as pl
from jax.experimental.pallas import tpu as pltpu
from jax.experimental.pallas import tpu_sc as plsc

B, N, DIM = 16, 1024, 256
BN = B * N
K = 1024
M = 512
NB = BN // M

NC, NS = 2, 16
NW = NC * NS
BPW = BN // NW
CH = 128
NCH = BPW // CH


def _tc_body(x_ref, e_ref, dist_ref, idx_ref):
    x = x_ref[...]
    e = e_ref[...]
    cross = jax.lax.dot_general(
        x, e, (((1,), (1,)), ((), ())), preferred_element_type=jnp.float32
    )
    x_sq = jnp.sum(x * x, axis=1, keepdims=True)
    e_sq = jnp.sum(e * e, axis=1)[None, :]
    dist = x_sq + e_sq - 2.0 * cross
    dist_ref[...] = dist
    idx = jnp.argmin(dist, axis=1).astype(jnp.int32)
    idx_ref[...] = idx.reshape(1, 1, M)


@functools.partial(
    pl.kernel,
    mesh=plsc.VectorSubcoreMesh(core_axis_name="c", subcore_axis_name="s"),
    out_type=jax.ShapeDtypeStruct((BN, DIM), jnp.float32),
    scratch_types=[
        pltpu.VMEM((BPW,), jnp.int32),
        pltpu.VMEM((CH, DIM), jnp.float32),
        pltpu.VMEM((CH, DIM), jnp.float32),
        pltpu.SemaphoreType.DMA,
        pltpu.SemaphoreType.DMA,
        pltpu.SemaphoreType.DMA,
        pltpu.SemaphoreType.DMA,
    ],
)
def _sc_gather(idx_hbm, e_hbm, out_hbm, idx_v, r0, r1, g0, g1, w0, w1):
    wid = lax.axis_index("s") * NC + lax.axis_index("c")
    base = wid * BPW
    pltpu.sync_copy(idx_hbm.at[pl.ds(base, BPW)], idx_v)
    rows = (r0, r1)
    gsem = (g0, g1)
    wsem = (w0, w1)
    gathers = [None] * NCH
    writes = [None] * NCH
    gathers[0] = pltpu.async_copy(
        e_hbm.at[idx_v.at[pl.ds(0, CH)]], rows[0], gsem[0]
    )
    for c in range(NCH):
        if c + 1 < NCH:
            if c - 1 >= 0:
                writes[c - 1].wait()
            gathers[c + 1] = pltpu.async_copy(
                e_hbm.at[idx_v.at[pl.ds((c + 1) * CH, CH)]],
                rows[(c + 1) % 2],
                gsem[(c + 1) % 2],
            )
        gathers[c].wait()
        writes[c] = pltpu.async_copy(
            rows[c % 2], out_hbm.at[pl.ds(base + c * CH, CH)], wsem[c % 2]
        )
    writes[NCH - 2].wait()
    writes[NCH - 1].wait()


def kernel(x, embed):
    xf = x.reshape(BN, DIM)
    e = embed.reshape(K, DIM)
    dist, idx3 = pl.pallas_call(
        _tc_body,
        grid=(NB,),
        in_specs=[
            pl.BlockSpec((M, DIM), lambda i: (i, 0)),
            pl.BlockSpec((K, DIM), lambda i: (0, 0)),
        ],
        out_specs=[
            pl.BlockSpec((M, K), lambda i: (i, 0)),
            pl.BlockSpec((1, 1, M), lambda i: (i, 0, 0)),
        ],
        out_shape=[
            jax.ShapeDtypeStruct((BN, K), jnp.float32),
            jax.ShapeDtypeStruct((NB, 1, M), jnp.int32),
        ],
    )(xf, e)
    idx = idx3.reshape(BN)
    q = _sc_gather(idx, e)
    return q.reshape(BN, 1, DIM), idx, dist


# speedup vs baseline: 1.1219x; 1.1112x over previous
import functools

import jax
import jax.numpy as jnp
from jax import lax
from jax.experimental import pallas as pl
from jax.experimental.pallas import tpu as pltpu
from jax.experimental.pallas import tpu_sc as plsc

B, N, DIM = 16, 1024, 256
BN = B * N
K = 1024
M = 1024
NB = BN // M

NC, NS = 2, 16
NW = NC * NS
BPW = BN // NW
CH = 128
NCH = BPW // CH


def _tc_body(x_ref, e_ref, dist_ref, idx_ref, esq_ref):
    @pl.when(pl.program_id(0) == 0)
    def _():
        e0 = e_ref[...]
        esq_ref[...] = jnp.sum(e0 * e0, axis=1)[None, :]

    x = x_ref[...]
    e = e_ref[...]
    cross = jax.lax.dot_general(
        x, e, (((1,), (1,)), ((), ())), preferred_element_type=jnp.float32
    )
    x_sq = jnp.sum(x * x, axis=1, keepdims=True)
    dist = x_sq + esq_ref[...] - 2.0 * cross
    dist_ref[...] = dist
    m = jnp.min(dist, axis=1, keepdims=True)
    kiota = jax.lax.broadcasted_iota(jnp.int32, (M, K), 1)
    masked = jnp.where(dist == m, kiota, K)
    idx = jnp.min(masked, axis=1).astype(jnp.int32)
    idx_ref[...] = idx.reshape(1, 8, 128)


@functools.partial(
    pl.kernel,
    mesh=plsc.VectorSubcoreMesh(core_axis_name="c", subcore_axis_name="s"),
    out_type=jax.ShapeDtypeStruct((BN, DIM), jnp.float32),
    scratch_types=[
        pltpu.VMEM((BPW,), jnp.int32),
        pltpu.VMEM((CH, DIM), jnp.float32),
        pltpu.VMEM((CH, DIM), jnp.float32),
        pltpu.SemaphoreType.DMA,
        pltpu.SemaphoreType.DMA,
        pltpu.SemaphoreType.DMA,
        pltpu.SemaphoreType.DMA,
    ],
)
def _sc_gather(idx_hbm, e_hbm, out_hbm, idx_v, r0, r1, g0, g1, w0, w1):
    wid = lax.axis_index("s") * NC + lax.axis_index("c")
    base = wid * BPW
    pltpu.sync_copy(idx_hbm.at[pl.ds(base, BPW)], idx_v)
    rows = (r0, r1)
    gsem = (g0, g1)
    wsem = (w0, w1)
    gathers = [None] * NCH
    writes = [None] * NCH
    gathers[0] = pltpu.async_copy(
        e_hbm.at[idx_v.at[pl.ds(0, CH)]], rows[0], gsem[0]
    )
    for c in range(NCH):
        if c + 1 < NCH:
            if c - 1 >= 0:
                writes[c - 1].wait()
            gathers[c + 1] = pltpu.async_copy(
                e_hbm.at[idx_v.at[pl.ds((c + 1) * CH, CH)]],
                rows[(c + 1) % 2],
                gsem[(c + 1) % 2],
            )
        gathers[c].wait()
        writes[c] = pltpu.async_copy(
            rows[c % 2], out_hbm.at[pl.ds(base + c * CH, CH)], wsem[c % 2]
        )
    writes[NCH - 2].wait()
    writes[NCH - 1].wait()


def kernel(x, embed):
    xf = x.reshape(BN, DIM)
    e = embed.reshape(K, DIM)
    dist, idx3 = pl.pallas_call(
        _tc_body,
        grid=(NB,),
        in_specs=[
            pl.BlockSpec((M, DIM), lambda i: (i, 0)),
            pl.BlockSpec((K, DIM), lambda i: (0, 0)),
        ],
        out_specs=[
            pl.BlockSpec((M, K), lambda i: (i, 0)),
            pl.BlockSpec((1, 8, 128), lambda i: (i, 0, 0)),
        ],
        out_shape=[
            jax.ShapeDtypeStruct((BN, K), jnp.float32),
            jax.ShapeDtypeStruct((NB, 8, 128), jnp.int32),
        ],
        scratch_shapes=[pltpu.VMEM((1, K), jnp.float32)],
    )(xf, e)
    idx = idx3.reshape(BN)
    q = _sc_gather(idx, e)
    return q.reshape(BN, 1, DIM), idx, dist
